# Initial kernel scaffold; baseline (speedup 1.0000x reference)
import jax
import jax.numpy as jnp
from jax.experimental import pallas as pl


def kernel(x, edge_index, edge_attr, W):
    raise NotImplementedError("write your pallas kernel here")


# speedup vs baseline: 2.5104x; 2.5104x over previous
import functools

import jax
import jax.numpy as jnp
from jax import lax
from jax.experimental import pallas as pl
from jax.experimental.pallas import tpu as pltpu
from jax.experimental.pallas import tpu_sc as plsc

_NC = 2
_NS = 16
_CH = 128
_GR = 8


def _acc_plan(n):
    nacc = ((n + 1 + 127) // 128) * 128
    zrpt = nacc // _NS
    rpt = (n // _NS) & ~7
    rpt_last = n - rpt * (_NS - 1)
    return nacc, zrpt, rpt, rpt_last


def _zero_acc(z_hbm, stage_v, acc, zr0, zrpt):
    pltpu.sync_copy(z_hbm, stage_v)
    zfull, zrem = divmod(zrpt, _CH)
    for k in range(zfull):
        pltpu.sync_copy(stage_v, acc.at[pl.ds(zr0 + _CH * k, _CH)])
    if zrem:
        pltpu.sync_copy(stage_v.at[pl.ds(0, zrem)],
                        acc.at[pl.ds(zr0 + _CH * zfull, zrem)])


@functools.partial(jax.jit, static_argnames=("n", "epad", "dfh"))
def _sc_segsum_x(x_pair, srcb, dstb, n, epad, dfh):
    nacc, zrpt, rpt, rpt_last = _acc_plan(n)
    gpt = epad // (_CH * _GR * _NS)
    mesh = plsc.VectorSubcoreMesh(core_axis_name="c", subcore_axis_name="s")

    @functools.partial(
        pl.kernel,
        out_type=jax.ShapeDtypeStruct((_NC, n, dfh), jnp.float32),
        mesh=mesh,
        scratch_types=[
            pltpu.VMEM((_GR, _CH), jnp.int32),
            pltpu.VMEM((_GR, _CH), jnp.int32),
            pltpu.VMEM((_CH, dfh), jnp.float32),
            pltpu.SemaphoreType.DMA,
            pltpu.VMEM_SHARED((nacc, dfh), jnp.float32),
        ],
    )
    def sc_kernel(xp_hbm, srcb_hbm, dstb_hbm, z_hbm, s_out,
                  src_v, dst_v, rows_v, gsem, acc):
        c = lax.axis_index("c")
        s = lax.axis_index("s")
        _zero_acc(z_hbm, rows_v, acc, s * zrpt, zrpt)
        plsc.subcore_barrier()

        def group(g, carry):
            pltpu.sync_copy(srcb_hbm.at[c, g], src_v)
            pltpu.sync_copy(dstb_hbm.at[g], dst_v)
            for j in range(_GR):
                pltpu.async_copy(xp_hbm.at[src_v.at[j]], rows_v, gsem).wait()
                pltpu.sync_copy(rows_v, acc.at[dst_v.at[j]], add=True)
            return carry

        lax.fori_loop(s * gpt, (s + 1) * gpt, group, 0)
        plsc.subcore_barrier()

        def write_rows(row0, nrows):
            pltpu.sync_copy(acc.at[pl.ds(row0, nrows)],
                            s_out.at[c, pl.ds(row0, nrows)])

        @pl.when(s < _NS - 1)
        def _():
            write_rows(s * rpt, rpt)

        @pl.when(s == _NS - 1)
        def _():
            write_rows((_NS - 1) * rpt, rpt_last)

    zeros = jnp.zeros((_CH, dfh), jnp.float32)
    return sc_kernel(x_pair, srcb, dstb, zeros)


@functools.partial(jax.jit, static_argnames=("n", "epad", "dfh"))
def _sc_segsum_ea(ea128, dstb, n, epad, dfh):
    nacc, zrpt, rpt, rpt_last = _acc_plan(n)
    gpw = epad // (_CH * _GR * _NS * _NC)
    mesh = plsc.VectorSubcoreMesh(core_axis_name="c", subcore_axis_name="s")

    @functools.partial(
        pl.kernel,
        out_type=jax.ShapeDtypeStruct((_NC, n, dfh), jnp.float32),
        mesh=mesh,
        scratch_types=[
            pltpu.VMEM((_GR, _CH), jnp.int32),
            pltpu.VMEM((_CH, dfh), jnp.float32),
            pltpu.VMEM_SHARED((nacc, dfh), jnp.float32),
        ],
    )
    def sc_kernel(ea_hbm, dstb_hbm, z_hbm, t_out, dst_v, rows_v, acc):
        c = lax.axis_index("c")
        s = lax.axis_index("s")
        _zero_acc(z_hbm, rows_v, acc, s * zrpt, zrpt)
        plsc.subcore_barrier()

        def group(g, carry):
            pltpu.sync_copy(dstb_hbm.at[g], dst_v)
            for j in range(_GR):
                e0 = (g * _GR + j) * _CH
                pltpu.sync_copy(ea_hbm.at[pl.ds(e0, _CH)], rows_v)
                pltpu.sync_copy(rows_v, acc.at[dst_v.at[j]], add=True)
            return carry

        w = c * _NS + s
        lax.fori_loop(w * gpw, (w + 1) * gpw, group, 0)
        plsc.subcore_barrier()

        def write_rows(row0, nrows):
            pltpu.sync_copy(acc.at[pl.ds(row0, nrows)],
                            t_out.at[c, pl.ds(row0, nrows)])

        @pl.when(s < _NS - 1)
        def _():
            write_rows(s * rpt, rpt)

        @pl.when(s == _NS - 1)
        def _():
            write_rows((_NS - 1) * rpt, rpt_last)

    zeros = jnp.zeros((_CH, dfh), jnp.float32)
    return sc_kernel(ea128, dstb, zeros)


def _mm_body(s_ref, t_ref, wx0_ref, wx1_ref, we_ref, o_ref):
    acc = jnp.dot(s_ref[0], wx0_ref[...], preferred_element_type=jnp.float32)
    acc += jnp.dot(s_ref[1], wx1_ref[...], preferred_element_type=jnp.float32)
    acc += jnp.dot(t_ref[0] + t_ref[1], we_ref[...],
                   preferred_element_type=jnp.float32)
    o_ref[...] = acc


def kernel(x, edge_index, edge_attr, W):
    n, df = x.shape
    e = edge_index.shape[1]
    de = edge_attr.shape[1]
    do = W.shape[1]
    dfh = df // 2
    assert df == 2 * dfh and n % _NS == 0 and de <= dfh

    grp = _NC * _NS * _GR * _CH
    epad = ((e + grp - 1) // grp) * grp
    pad = epad - e
    src = edge_index[0]
    dst = edge_index[1]
    if pad:
        src = jnp.concatenate([src, jnp.zeros((pad,), jnp.int32)])
        dst = jnp.concatenate([dst, jnp.full((pad,), n, jnp.int32)])
    ngrp = epad // (_GR * _CH)
    srcb = jnp.stack([src, src + n]).reshape(2, ngrp, _GR, _CH)
    dstb = dst.reshape(ngrp, _GR, _CH)

    x_pair = jnp.concatenate([x[:, :dfh], x[:, dfh:]], axis=0)
    ea128 = jnp.zeros((epad, dfh), jnp.float32).at[:e, :de].set(edge_attr)
    we128 = jnp.zeros((dfh, do), jnp.float32).at[:de].set(W[df:])

    s_acc = _sc_segsum_x(x_pair, srcb, dstb, n=n, epad=epad, dfh=dfh)
    t_acc = _sc_segsum_ea(ea128, dstb, n=n, epad=epad, dfh=dfh)

    mb = 1000
    out = pl.pallas_call(
        _mm_body,
        grid=(n // mb,),
        in_specs=[
            pl.BlockSpec((_NC, mb, dfh), lambda i: (0, i, 0)),
            pl.BlockSpec((_NC, mb, dfh), lambda i: (0, i, 0)),
            pl.BlockSpec((dfh, do), lambda i: (0, 0)),
            pl.BlockSpec((dfh, do), lambda i: (0, 0)),
            pl.BlockSpec((dfh, do), lambda i: (0, 0)),
        ],
        out_specs=pl.BlockSpec((mb, do), lambda i: (i, 0)),
        out_shape=jax.ShapeDtypeStruct((n, do), jnp.float32),
    )(s_acc, t_acc, W[:dfh], W[dfh:df], we128)
    return out


# speedup vs baseline: 2.9172x; 1.1621x over previous
import functools

import jax
import jax.numpy as jnp
from jax import lax
from jax.experimental import pallas as pl
from jax.experimental.pallas import tpu as pltpu
from jax.experimental.pallas import tpu_sc as plsc

_NC = 2
_NS = 16
_CH = 128
_GR = 8


def _acc_plan(n):
    nacc = ((n + 1 + 127) // 128) * 128
    zrpt = nacc // _NS
    rpt = (n // _NS) & ~7
    rpt_last = n - rpt * (_NS - 1)
    return nacc, zrpt, rpt, rpt_last


def _zero_acc(z_hbm, stage_v, acc, zr0, zrpt):
    pltpu.sync_copy(z_hbm, stage_v)
    zfull, zrem = divmod(zrpt, _CH)
    for k in range(zfull):
        pltpu.sync_copy(stage_v, acc.at[pl.ds(zr0 + _CH * k, _CH)])
    if zrem:
        pltpu.sync_copy(stage_v.at[pl.ds(0, zrem)],
                        acc.at[pl.ds(zr0 + _CH * zfull, zrem)])


@functools.partial(jax.jit, static_argnames=("n", "epad", "dfh"))
def _sc_segsum_x(x_pair, srcb, dstb, n, epad, dfh):
    nacc, zrpt, rpt, rpt_last = _acc_plan(n)
    gpt = epad // (_CH * _GR * _NS)
    mesh = plsc.VectorSubcoreMesh(core_axis_name="c", subcore_axis_name="s")

    @functools.partial(
        pl.kernel,
        out_type=jax.ShapeDtypeStruct((_NC, n, dfh), jnp.float32),
        mesh=mesh,
        scratch_types=[
            pltpu.VMEM((_GR, _CH), jnp.int32),
            pltpu.VMEM((_GR, _CH), jnp.int32),
            pltpu.VMEM((_CH, dfh), jnp.float32),
            pltpu.VMEM((_CH, dfh), jnp.float32),
            pltpu.SemaphoreType.DMA,
            pltpu.SemaphoreType.DMA,
            pltpu.SemaphoreType.DMA,
            pltpu.SemaphoreType.DMA,
            pltpu.VMEM_SHARED((nacc, dfh), jnp.float32),
        ],
    )
    def sc_kernel(xp_hbm, srcb_hbm, dstb_hbm, z_hbm, s_out,
                  src_v, dst_v, rows_a, rows_b, gs0, gs1, ss0, ss1, acc):
        c = lax.axis_index("c")
        s = lax.axis_index("s")
        _zero_acc(z_hbm, rows_a, acc, s * zrpt, zrpt)
        plsc.subcore_barrier()

        bufs = (rows_a, rows_b)
        gsems = (gs0, gs1)
        ssems = (ss0, ss1)

        def group(g, carry):
            pltpu.sync_copy(srcb_hbm.at[c, g], src_v)
            pltpu.sync_copy(dstb_hbm.at[g], dst_v)
            gd = [None, None]
            sd = [None, None]
            gd[0] = pltpu.async_copy(xp_hbm.at[src_v.at[0]], bufs[0], gsems[0])
            for j in range(_GR):
                p = j % 2
                if j + 1 < _GR:
                    if j >= 1:
                        sd[1 - p].wait()
                    gd[1 - p] = pltpu.async_copy(
                        xp_hbm.at[src_v.at[j + 1]], bufs[1 - p], gsems[1 - p])
                gd[p].wait()
                sd[p] = pltpu.async_copy(
                    bufs[p], acc.at[dst_v.at[j]], ssems[p], add=True)
            sd[0].wait()
            sd[1].wait()
            return carry

        lax.fori_loop(s * gpt, (s + 1) * gpt, group, 0)
        plsc.subcore_barrier()

        def write_rows(row0, nrows):
            pltpu.sync_copy(acc.at[pl.ds(row0, nrows)],
                            s_out.at[c, pl.ds(row0, nrows)])

        @pl.when(s < _NS - 1)
        def _():
            write_rows(s * rpt, rpt)

        @pl.when(s == _NS - 1)
        def _():
            write_rows((_NS - 1) * rpt, rpt_last)

    zeros = jnp.zeros((_CH, dfh), jnp.float32)
    return sc_kernel(x_pair, srcb, dstb, zeros)


@functools.partial(jax.jit, static_argnames=("n", "epad", "dfh"))
def _sc_segsum_ea(ea128, dstb, n, epad, dfh):
    nacc, zrpt, rpt, rpt_last = _acc_plan(n)
    gpw = epad // (_CH * _GR * _NS * _NC)
    mesh = plsc.VectorSubcoreMesh(core_axis_name="c", subcore_axis_name="s")

    @functools.partial(
        pl.kernel,
        out_type=jax.ShapeDtypeStruct((_NC, n, dfh), jnp.float32),
        mesh=mesh,
        scratch_types=[
            pltpu.VMEM((_GR, _CH), jnp.int32),
            pltpu.VMEM((_CH, dfh), jnp.float32),
            pltpu.VMEM((_CH, dfh), jnp.float32),
            pltpu.SemaphoreType.DMA,
            pltpu.SemaphoreType.DMA,
            pltpu.SemaphoreType.DMA,
            pltpu.SemaphoreType.DMA,
            pltpu.VMEM_SHARED((nacc, dfh), jnp.float32),
        ],
    )
    def sc_kernel(ea_hbm, dstb_hbm, z_hbm, t_out,
                  dst_v, rows_a, rows_b, gs0, gs1, ss0, ss1, acc):
        c = lax.axis_index("c")
        s = lax.axis_index("s")
        _zero_acc(z_hbm, rows_a, acc, s * zrpt, zrpt)
        plsc.subcore_barrier()

        bufs = (rows_a, rows_b)
        gsems = (gs0, gs1)
        ssems = (ss0, ss1)

        def group(g, carry):
            pltpu.sync_copy(dstb_hbm.at[g], dst_v)
            e0 = g * _GR * _CH
            gd = [None, None]
            sd = [None, None]
            gd[0] = pltpu.async_copy(ea_hbm.at[pl.ds(e0, _CH)], bufs[0],
                                     gsems[0])
            for j in range(_GR):
                p = j % 2
                if j + 1 < _GR:
                    if j >= 1:
                        sd[1 - p].wait()
                    gd[1 - p] = pltpu.async_copy(
                        ea_hbm.at[pl.ds(e0 + (j + 1) * _CH, _CH)],
                        bufs[1 - p], gsems[1 - p])
                gd[p].wait()
                sd[p] = pltpu.async_copy(
                    bufs[p], acc.at[dst_v.at[j]], ssems[p], add=True)
            sd[0].wait()
            sd[1].wait()
            return carry

        w = c * _NS + s
        lax.fori_loop(w * gpw, (w + 1) * gpw, group, 0)
        plsc.subcore_barrier()

        def write_rows(row0, nrows):
            pltpu.sync_copy(acc.at[pl.ds(row0, nrows)],
                            t_out.at[c, pl.ds(row0, nrows)])

        @pl.when(s < _NS - 1)
        def _():
            write_rows(s * rpt, rpt)

        @pl.when(s == _NS - 1)
        def _():
            write_rows((_NS - 1) * rpt, rpt_last)

    zeros = jnp.zeros((_CH, dfh), jnp.float32)
    return sc_kernel(ea128, dstb, zeros)


def _mm_body(s_ref, t_ref, wx0_ref, wx1_ref, we_ref, o_ref):
    acc = jnp.dot(s_ref[0], wx0_ref[...], preferred_element_type=jnp.float32)
    acc += jnp.dot(s_ref[1], wx1_ref[...], preferred_element_type=jnp.float32)
    acc += jnp.dot(t_ref[0] + t_ref[1], we_ref[...],
                   preferred_element_type=jnp.float32)
    o_ref[...] = acc


def kernel(x, edge_index, edge_attr, W):
    n, df = x.shape
    e = edge_index.shape[1]
    de = edge_attr.shape[1]
    do = W.shape[1]
    dfh = df // 2
    assert df == 2 * dfh and n % _NS == 0 and de <= dfh

    grp = _NC * _NS * _GR * _CH
    epad = ((e + grp - 1) // grp) * grp
    pad = epad - e
    src = edge_index[0]
    dst = edge_index[1]
    if pad:
        src = jnp.concatenate([src, jnp.zeros((pad,), jnp.int32)])
        dst = jnp.concatenate([dst, jnp.full((pad,), n, jnp.int32)])
    ngrp = epad // (_GR * _CH)
    srcb = jnp.stack([src, src + n]).reshape(2, ngrp, _GR, _CH)
    dstb = dst.reshape(ngrp, _GR, _CH)

    x_pair = jnp.concatenate([x[:, :dfh], x[:, dfh:]], axis=0)
    ea128 = jnp.zeros((epad, dfh), jnp.float32).at[:e, :de].set(edge_attr)
    we128 = jnp.zeros((dfh, do), jnp.float32).at[:de].set(W[df:])

    s_acc = _sc_segsum_x(x_pair, srcb, dstb, n=n, epad=epad, dfh=dfh)
    t_acc = _sc_segsum_ea(ea128, dstb, n=n, epad=epad, dfh=dfh)

    mb = 1000
    out = pl.pallas_call(
        _mm_body,
        grid=(n // mb,),
        in_specs=[
            pl.BlockSpec((_NC, mb, dfh), lambda i: (0, i, 0)),
            pl.BlockSpec((_NC, mb, dfh), lambda i: (0, i, 0)),
            pl.BlockSpec((dfh, do), lambda i: (0, 0)),
            pl.BlockSpec((dfh, do), lambda i: (0, 0)),
            pl.BlockSpec((dfh, do), lambda i: (0, 0)),
        ],
        out_specs=pl.BlockSpec((mb, do), lambda i: (i, 0)),
        out_shape=jax.ShapeDtypeStruct((n, do), jnp.float32),
    )(s_acc, t_acc, W[:dfh], W[dfh:df], we128)
    return out
